# baseline (speedup 1.0000x reference)
import functools

import jax
import jax.numpy as jnp
from jax import lax
from jax.experimental import pallas as pl
from jax.experimental.pallas import tpu as pltpu
from jax.experimental.pallas import tpu_sc as plsc

B = 16384
N_NODES = 100000
MD = 128
ED = 16
TD = 100

NC, NS, L = 2, 16, 16
NW = NC * NS
TWO_B = 2 * B
PERW = TWO_B // NW
GC = 256
TBL = 3136
CT = 18080
CPW = (N_NODES - CT) // NW
CCH = 256
SCCH = 384
IDC = 2048


def _gather_body(mem_hbm, ids_hbm, out_hbm, idx0, idx1, r0, r1, s0, s1):
    wid = lax.axis_index("s") * NC + lax.axis_index("c")
    base = wid * PERW
    part = wid // (NW // 2)
    pbase = base - part * B
    nch = PERW // GC
    idx = (idx0, idx1)
    rows = (r0, r1)
    sems = (s0, s1)

    pltpu.sync_copy(ids_hbm.at[pl.ds(base, GC)], idx0)
    pltpu.async_copy(mem_hbm.at[idx0], r0, s0)
    for i in range(1, nch):
        b, pb = i % 2, (i - 1) % 2
        off = base + i * GC
        pltpu.sync_copy(ids_hbm.at[pl.ds(off, GC)], idx[b])
        pltpu.async_copy(mem_hbm.at[idx[b]], rows[b], sems[b])
        pltpu.make_async_copy(mem_hbm.at[idx[pb]], rows[pb], sems[pb]).wait()
        pltpu.sync_copy(rows[pb],
                        out_hbm.at[part].at[pl.ds(pbase + (i - 1) * GC, GC)])
    pb = (nch - 1) % 2
    pltpu.make_async_copy(mem_hbm.at[idx[pb]], rows[pb], sems[pb]).wait()
    pltpu.sync_copy(rows[pb],
                    out_hbm.at[part].at[pl.ds(pbase + (nch - 1) * GC, GC)])


def _sc_gather(memory, ids):
    mesh = plsc.VectorSubcoreMesh(core_axis_name="c", subcore_axis_name="s")
    fn = pl.kernel(
        _gather_body,
        out_type=jax.ShapeDtypeStruct((2, B, MD), jnp.float32),
        mesh=mesh,
        compiler_params=pltpu.CompilerParams(needs_layout_passes=False,
                                             use_tc_tiling_on_sc=True),
        scratch_types=[
            pltpu.VMEM((GC,), jnp.int32),
            pltpu.VMEM((GC,), jnp.int32),
            pltpu.VMEM((GC, MD), jnp.float32),
            pltpu.VMEM((GC, MD), jnp.float32),
            pltpu.SemaphoreType.DMA,
            pltpu.SemaphoreType.DMA,
        ],
    )
    return fn(memory, ids)


def _winners_body(ids_hbm, mem_hbm, base_ref, wl_hbm, cnt_hbm,
                  idc, wl, table, c0, c1, s0, s1, o0, o1):
    wid = lax.axis_index("s") * NC + lax.axis_index("c")
    iota = lax.iota(jnp.int32, L)
    cbase = CT + wid * CPW
    bufs = (c0, c1)
    insems = (s0, s1)
    outsems = (o0, o1)
    nco = CPW // CCH

    def cp_in(i):
        pltpu.async_copy(mem_hbm.at[pl.ds(cbase + i * CCH, CCH)],
                         bufs[i % 2], insems[i % 2])

    def cp_wait_in(i):
        pltpu.make_async_copy(mem_hbm.at[pl.ds(cbase + i * CCH, CCH)],
                              bufs[i % 2], insems[i % 2]).wait()

    def cp_out(i):
        pltpu.async_copy(bufs[i % 2], base_ref.at[pl.ds(cbase + i * CCH, CCH)],
                         outsems[i % 2])

    def cp_wait_out(i):
        pltpu.make_async_copy(bufs[i % 2],
                              base_ref.at[pl.ds(cbase + i * CCH, CCH)],
                              outsems[i % 2]).wait()

    off = jnp.int32(0)
    for c in range(TWO_B // IDC):
        if c < nco:
            if c >= 2:
                cp_wait_out(c - 2)
            cp_in(c)
        if 1 <= c <= nco:
            cp_wait_in(c - 1)
            cp_out(c - 1)
        pltpu.sync_copy(ids_hbm.at[pl.ds(c * IDC, IDC)], idc)

        def inner(k, off, c=c):
            v = idc[pl.ds(k * L, L)]
            m = (v & (NW - 1)) == wid
            pos = c * IDC + k * L + iota
            packed = ((v >> 5) << 15) | pos
            plsc.store_compressed(wl.at[pl.ds(off, L)], packed, mask=m)
            cntv = plsc.all_reduce_population_count(m)
            return off + cntv[0]

        off = lax.fori_loop(0, IDC // L, inner, off)
    n = off
    cp_wait_out(nco - 2)
    cp_wait_out(nco - 1)
    nv = lax.div(n + (L - 1), jnp.int32(L))

    def initk(k, carry):
        valid = (k * L + iota) < n
        v = wl[pl.ds(k * L, L)]
        vr_s = jnp.where(valid, v >> 15, 0)
        plsc.store_scatter(table, [vr_s], v & 32767, mask=valid)
        return carry

    lax.fori_loop(0, nv, initk, 0)

    def round_body(state):
        def rk(k, cnt):
            valid = (k * L + iota) < n
            v = wl[pl.ds(k * L, L)]
            vp = v & 32767
            vr_s = jnp.where(valid, v >> 15, 0)
            t = plsc.load_gather(table, [vr_s])
            m2 = valid & (vp > t)
            plsc.store_scatter(table, [vr_s], vp, mask=m2)
            return cnt + jnp.sum(m2.astype(jnp.int32))

        c = lax.fori_loop(0, nv, rk, jnp.int32(0))
        return (c, state[1] + 1)

    lax.while_loop(lambda s: s[0] > 0, round_body, (jnp.int32(1), jnp.int32(0)))

    def wk(k, woff):
        valid = (k * L + iota) < n
        v = wl[pl.ds(k * L, L)]
        vp = v & 32767
        vr_s = jnp.where(valid, v >> 15, 0)
        t = plsc.load_gather(table, [vr_s])
        w = valid & (t == vp)
        wi = w.astype(jnp.int32)
        slots = woff + plsc.cumsum(wi) - 1
        slots = jnp.where(w, slots, 0)
        plsc.store_scatter(wl, [slots], v, mask=w)
        return woff + jnp.sum(wi)

    m = lax.fori_loop(0, nv, wk, jnp.int32(0))

    mpad = lax.div(m + (SCCH - 1), jnp.int32(SCCH)) * SCCH

    @pl.when(m > 0)
    def _pad():
        def padk(k, carry):
            idxs = m + k * L + iota
            mk = idxs < mpad
            src = idxs % m
            vals = plsc.load_gather(wl, [src])
            idxs_s = jnp.where(mk, idxs, 0)
            plsc.store_scatter(wl, [idxs_s], vals, mask=mk)
            return carry

        lax.fori_loop(0, SCCH // L, padk, 0)

    def wout(i, carry):
        pltpu.sync_copy(wl.at[pl.ds(i * IDC, IDC)],
                        wl_hbm.at[wid].at[pl.ds(i * IDC, IDC)])
        return carry

    lax.fori_loop(0, (mpad + (IDC - 1)) // IDC, wout, 0)
    for j in range(MD // L):
        idc[pl.ds(j * L, L)] = jnp.broadcast_to(mpad, (L,))
    pltpu.sync_copy(idc.at[pl.ds(0, MD)], cnt_hbm.at[wid])


def _sc_winners(ids, memory, base_ref):
    mesh = plsc.VectorSubcoreMesh(core_axis_name="c", subcore_axis_name="s")
    fn = pl.kernel(
        _winners_body,
        out_type=(
            jax.ShapeDtypeStruct((NW, TWO_B), jnp.int32),
            jax.ShapeDtypeStruct((NW, MD), jnp.int32),
        ),
        mesh=mesh,
        compiler_params=pltpu.CompilerParams(needs_layout_passes=False),
        scratch_types=[
            pltpu.VMEM((IDC,), jnp.int32),
            pltpu.VMEM((TWO_B + L,), jnp.int32),
            pltpu.VMEM((TBL,), jnp.int32),
            pltpu.VMEM((CCH, MD), jnp.float32),
            pltpu.VMEM((CCH, MD), jnp.float32),
            pltpu.SemaphoreType.DMA,
            pltpu.SemaphoreType.DMA,
            pltpu.SemaphoreType.DMA,
            pltpu.SemaphoreType.DMA,
        ],
    )
    return fn(ids, memory, base_ref)


def _apply_body(base_hbm, upd_hbm, wl_hbm, cnt_hbm,
                cbuf, wbuf, sg0, sp0, r0, sg1, sp1, r1, sem0, sem1):
    wid = lax.axis_index("s") * NC + lax.axis_index("c")
    pltpu.sync_copy(cnt_hbm.at[wid], cbuf)
    cv = cbuf[pl.ds(0, L)]
    mpad = cv[0]
    nch = mpad // SCCH

    def stage_and_fire(c, sg, sp, rows, sem):
        pltpu.sync_copy(wl_hbm.at[wid].at[pl.ds(c * SCCH, SCCH)], wbuf)

        def upk(k, carry):
            v = wbuf[pl.ds(k * L, L)]
            sp[pl.ds(k * L, L)] = v & 32767
            sg[pl.ds(k * L, L)] = ((v >> 15) << 5) | wid
            return carry

        lax.fori_loop(0, SCCH // L, upk, 0)
        pltpu.async_copy(upd_hbm.at[sp], rows, sem)

    def drain_and_scatter(sg, sp, rows, sem):
        pltpu.make_async_copy(upd_hbm.at[sp], rows, sem).wait()
        pltpu.async_copy(rows, base_hbm.at[sg], sem).wait()

    @pl.when(nch > 0)
    def _go():
        stage_and_fire(0, sg0, sp0, r0, sem0)

        def sck(c, carry):
            @pl.when(c % 2 == 0)
            def _even():
                @pl.when(c + 1 < nch)
                def _pf():
                    stage_and_fire(c + 1, sg1, sp1, r1, sem1)
                drain_and_scatter(sg0, sp0, r0, sem0)

            @pl.when(c % 2 == 1)
            def _odd():
                @pl.when(c + 1 < nch)
                def _pf():
                    stage_and_fire(c + 1, sg0, sp0, r0, sem0)
                drain_and_scatter(sg1, sp1, r1, sem1)

            return carry

        lax.fori_loop(0, nch, sck, 0)


def _sc_apply(base_ref, upds, wl, cnt):
    mesh = plsc.VectorSubcoreMesh(core_axis_name="c", subcore_axis_name="s")
    fn = pl.kernel(
        _apply_body,
        out_type=(),
        mesh=mesh,
        compiler_params=pltpu.CompilerParams(needs_layout_passes=False),
        scratch_types=[
            pltpu.VMEM((MD,), jnp.int32),
            pltpu.VMEM((SCCH,), jnp.int32),
            pltpu.VMEM((SCCH,), jnp.int32),
            pltpu.VMEM((SCCH,), jnp.int32),
            pltpu.VMEM((SCCH, MD), jnp.float32),
            pltpu.VMEM((SCCH,), jnp.int32),
            pltpu.VMEM((SCCH,), jnp.int32),
            pltpu.VMEM((SCCH, MD), jnp.float32),
            pltpu.SemaphoreType.DMA,
            pltpu.SemaphoreType.DMA,
        ],
    )
    fn(base_ref, upds, wl, cnt)


def _dense_body(g_ref, se_ref, de_ref, ef_ref, ts_ref, tw_ref, tb_ref,
                evm_ref, evt_ref, as_ref, ao_ref, ae_ref, at_ref,
                ca_ref, ce_ref, ct_ref, cw2_ref, ou_ref, on_ref,
                eb_ref, ab_ref, c1b_ref, c2b_ref, ob_ref,
                out_ref, upd_ref):
    f32 = jnp.float32
    bf16 = jnp.bfloat16

    def dot(a, b):
        return lax.dot_general(a.astype(bf16), b, (((1,), (0,)), ((), ())),
                               preferred_element_type=f32)

    te = jnp.cos(ts_ref[...] * tw_ref[...] + tb_ref[...])
    ef = ef_ref[...]
    sm = g_ref[0]
    dm = g_ref[1]

    sh_e = dot(te, evt_ref[...]) + eb_ref[...]
    s_ev = jnp.tanh(dot(sm, evm_ref[...]) + sh_e)
    d_ev = jnp.tanh(dot(dm, evm_ref[...]) + sh_e)

    sh_a = dot(ef, ae_ref[...]) + dot(te, at_ref[...]) + ab_ref[...]
    s_as = jnp.tanh(dot(s_ev, as_ref[...]) + dot(d_ev, ao_ref[...]) + sh_a)
    d_as = jnp.tanh(dot(d_ev, as_ref[...]) + dot(s_ev, ao_ref[...]) + sh_a)

    sh_c = dot(ef, ce_ref[...]) + dot(te, ct_ref[...]) + c1b_ref[...]
    s_c1 = jnp.maximum(dot(s_as, ca_ref[...]) + sh_c, 0.0)
    d_c1 = jnp.maximum(dot(d_as, ca_ref[...]) + sh_c, 0.0)
    s_cm = jnp.tanh(dot(s_c1, cw2_ref[...]) + c2b_ref[...])
    d_cm = jnp.tanh(dot(d_c1, cw2_ref[...]) + c2b_ref[...])

    u_s = s_ev + s_cm
    u_d = d_ev + d_cm
    upd_ref[0] = u_s
    upd_ref[1] = u_d
    out_ref[0] = dot(u_s, ou_ref[...]) + dot(se_ref[...], on_ref[...]) + ob_ref[...]
    out_ref[1] = dot(u_d, ou_ref[...]) + dot(de_ref[...], on_ref[...]) + ob_ref[...]


BLK = 2048


def _tc_dense(g3, semb, demb, ef, ts2, tw, tb, weights):
    f32 = jnp.float32
    grid = (B // BLK,)

    def full(shape):
        return pl.BlockSpec(shape, lambda g: tuple(0 for _ in shape))

    in_specs = [
        pl.BlockSpec((2, BLK, MD), lambda g: (0, g, 0)),
        pl.BlockSpec((BLK, MD), lambda g: (g, 0)),
        pl.BlockSpec((BLK, MD), lambda g: (g, 0)),
        pl.BlockSpec((BLK, ED), lambda g: (g, 0)),
        pl.BlockSpec((BLK, 1), lambda g: (g, 0)),
        full((1, MD)), full((1, MD)),
        full((MD, MD)), full((MD, MD)),
        full((MD, MD)), full((MD, MD)), full((ED, MD)), full((MD, MD)),
        full((MD, MD)), full((ED, MD)), full((MD, MD)), full((MD, MD)),
        full((MD, MD)), full((MD, MD)),
        full((1, MD)), full((1, MD)), full((1, MD)), full((1, MD)), full((1, MD)),
    ]
    out_specs = [
        pl.BlockSpec((2, BLK, MD), lambda g: (0, g, 0)),
        pl.BlockSpec((2, BLK, MD), lambda g: (0, g, 0)),
    ]
    out_shape = [
        jax.ShapeDtypeStruct((2, B, MD), f32),
        jax.ShapeDtypeStruct((2, B, MD), f32),
    ]
    return pl.pallas_call(
        _dense_body,
        grid=grid,
        in_specs=in_specs,
        out_specs=out_specs,
        out_shape=out_shape,
    )(g3, semb, demb, ef, ts2, tw, tb, *weights)


CPB = 4520


def _copy_body(src_ref, dst_ref):
    dst_ref[...] = src_ref[...]


def _tc_copy(memory):
    return pl.pallas_call(
        _copy_body,
        grid=(CT // CPB,),
        in_specs=[pl.BlockSpec((CPB, MD), lambda g: (g, 0))],
        out_specs=pl.BlockSpec((CPB, MD), lambda g: (g, 0)),
        out_shape=jax.ShapeDtypeStruct((N_NODES, MD), jnp.float32),
    )(memory)


def kernel(src_node_embeddings, dst_node_embeddings, src_node_ids,
           dst_node_ids, edge_features, timestamps, memory,
           time_w, time_b, evo_w, evo_b, assoc_w, assoc_b,
           comm_w1, comm_b1, comm_w2, comm_b2, out_w, out_b):
    f32 = jnp.float32
    ids = jnp.concatenate([src_node_ids.astype(jnp.int32),
                           dst_node_ids.astype(jnp.int32)])

    g3 = _sc_gather(memory, ids)

    ts2 = timestamps.reshape(B, 1)
    pad_t = MD - TD
    tw = jnp.pad(time_w, (0, pad_t)).reshape(1, MD)
    tb = jnp.pad(time_b, (0, pad_t)).reshape(1, MD)
    evm = evo_w[:MD]
    evt = jnp.pad(evo_w[MD:], ((0, pad_t), (0, 0)))
    a_s = assoc_w[:MD]
    a_o = assoc_w[MD:2 * MD]
    a_e = assoc_w[2 * MD:2 * MD + ED]
    a_t = jnp.pad(assoc_w[2 * MD + ED:], ((0, pad_t), (0, 0)))
    c_a = comm_w1[:MD]
    c_e = comm_w1[MD:MD + ED]
    c_t = jnp.pad(comm_w1[MD + ED:], ((0, pad_t), (0, 0)))
    o_u = out_w[:MD]
    o_n = out_w[MD:]
    bf16 = jnp.bfloat16
    weights = tuple(w.astype(bf16) for w in
                    (evm, evt, a_s, a_o, a_e, a_t, c_a, c_e, c_t, comm_w2,
                     o_u, o_n)) + (
               evo_b.reshape(1, MD), assoc_b.reshape(1, MD),
               comm_b1.reshape(1, MD), comm_b2.reshape(1, MD),
               out_b.reshape(1, MD))

    base = _tc_copy(memory)
    base_ref = jax.new_ref(base)
    wl, cnt = _sc_winners(ids, memory, base_ref)
    outp3, upd3 = _tc_dense(g3, src_node_embeddings, dst_node_embeddings,
                            edge_features, ts2, tw, tb, weights)

    output = outp3.reshape(TWO_B, MD)
    upds = upd3.reshape(TWO_B, MD)
    _sc_apply(base_ref, upds, wl, cnt)
    new_memory = base_ref[...]
    return output, new_memory


# speedup vs baseline: 1.0052x; 1.0052x over previous
import jax
import jax.numpy as jnp
from jax import lax
from jax.experimental import pallas as pl
from jax.experimental.pallas import tpu as pltpu
from jax.experimental.pallas import tpu_sc as plsc

B = 16384
N_NODES = 100000
MD = 128
ED = 16
TD = 100

NC, NS, L = 2, 16, 16
NW = NC * NS
TWO_B = 2 * B
PERW = TWO_B // NW
GC = 256
TBL = 3136
CT = 18080
CPW = (N_NODES - CT) // NW
CCH = 256
SCCH = 384
IDC = 2048


def _gather_body(mem_hbm, ids_hbm, out_hbm, idx0, idx1, r0, r1, s0, s1):
    wid = lax.axis_index("s") * NC + lax.axis_index("c")
    base = wid * PERW
    part = wid // (NW // 2)
    pbase = base - part * B
    nch = PERW // GC
    idx = (idx0, idx1)
    rows = (r0, r1)
    sems = (s0, s1)

    pltpu.sync_copy(ids_hbm.at[pl.ds(base, GC)], idx0)
    pltpu.async_copy(mem_hbm.at[idx0], r0, s0)
    for i in range(1, nch):
        b, pb = i % 2, (i - 1) % 2
        off = base + i * GC
        pltpu.sync_copy(ids_hbm.at[pl.ds(off, GC)], idx[b])
        pltpu.async_copy(mem_hbm.at[idx[b]], rows[b], sems[b])
        pltpu.make_async_copy(mem_hbm.at[idx[pb]], rows[pb], sems[pb]).wait()
        pltpu.sync_copy(rows[pb],
                        out_hbm.at[part].at[pl.ds(pbase + (i - 1) * GC, GC)])
    pb = (nch - 1) % 2
    pltpu.make_async_copy(mem_hbm.at[idx[pb]], rows[pb], sems[pb]).wait()
    pltpu.sync_copy(rows[pb],
                    out_hbm.at[part].at[pl.ds(pbase + (nch - 1) * GC, GC)])


def _sc_gather(memory, ids):
    mesh = plsc.VectorSubcoreMesh(core_axis_name="c", subcore_axis_name="s")
    fn = pl.kernel(
        _gather_body,
        out_type=jax.ShapeDtypeStruct((2, B, MD), jnp.float32),
        mesh=mesh,
        compiler_params=pltpu.CompilerParams(needs_layout_passes=False),
        scratch_types=[
            pltpu.VMEM((GC,), jnp.int32),
            pltpu.VMEM((GC,), jnp.int32),
            pltpu.VMEM((GC, MD), jnp.float32),
            pltpu.VMEM((GC, MD), jnp.float32),
            pltpu.SemaphoreType.DMA,
            pltpu.SemaphoreType.DMA,
        ],
    )
    return fn(memory, ids)


def _winners_body(ids_hbm, mem_hbm, base_ref, wl_hbm, cnt_hbm,
                  idc, wl, table, c0, c1, s0, s1, o0, o1):
    wid = lax.axis_index("s") * NC + lax.axis_index("c")
    iota = lax.iota(jnp.int32, L)
    cbase = CT + wid * CPW
    bufs = (c0, c1)
    insems = (s0, s1)
    outsems = (o0, o1)
    nco = CPW // CCH

    def cp_in(i):
        pltpu.async_copy(mem_hbm.at[pl.ds(cbase + i * CCH, CCH)],
                         bufs[i % 2], insems[i % 2])

    def cp_wait_in(i):
        pltpu.make_async_copy(mem_hbm.at[pl.ds(cbase + i * CCH, CCH)],
                              bufs[i % 2], insems[i % 2]).wait()

    def cp_out(i):
        pltpu.async_copy(bufs[i % 2], base_ref.at[pl.ds(cbase + i * CCH, CCH)],
                         outsems[i % 2])

    def cp_wait_out(i):
        pltpu.make_async_copy(bufs[i % 2],
                              base_ref.at[pl.ds(cbase + i * CCH, CCH)],
                              outsems[i % 2]).wait()

    off = jnp.int32(0)
    for c in range(TWO_B // IDC):
        if c < nco:
            if c >= 2:
                cp_wait_out(c - 2)
            cp_in(c)
        if 1 <= c <= nco:
            cp_wait_in(c - 1)
            cp_out(c - 1)
        pltpu.sync_copy(ids_hbm.at[pl.ds(c * IDC, IDC)], idc)

        def inner(k, off, c=c):
            v = idc[pl.ds(k * L, L)]
            m = (v & (NW - 1)) == wid
            pos = c * IDC + k * L + iota
            packed = ((v >> 5) << 15) | pos
            plsc.store_compressed(wl.at[pl.ds(off, L)], packed, mask=m)
            cntv = plsc.all_reduce_population_count(m)
            return off + cntv[0]

        off = lax.fori_loop(0, IDC // L, inner, off)
    n = off
    cp_wait_out(nco - 2)
    cp_wait_out(nco - 1)
    nv = lax.div(n + (L - 1), jnp.int32(L))

    def initk(k, carry):
        valid = (k * L + iota) < n
        v = wl[pl.ds(k * L, L)]
        vr_s = jnp.where(valid, v >> 15, 0)
        plsc.store_scatter(table, [vr_s], v & 32767, mask=valid)
        return carry

    lax.fori_loop(0, nv, initk, 0)

    def round_body(state):
        def rk(k, cnt):
            valid = (k * L + iota) < n
            v = wl[pl.ds(k * L, L)]
            vp = v & 32767
            vr_s = jnp.where(valid, v >> 15, 0)
            t = plsc.load_gather(table, [vr_s])
            m2 = valid & (vp > t)
            plsc.store_scatter(table, [vr_s], vp, mask=m2)
            return cnt + jnp.sum(m2.astype(jnp.int32))

        c = lax.fori_loop(0, nv, rk, jnp.int32(0))
        return (c, state[1] + 1)

    lax.while_loop(lambda s: s[0] > 0, round_body, (jnp.int32(1), jnp.int32(0)))

    def wk(k, woff):
        valid = (k * L + iota) < n
        v = wl[pl.ds(k * L, L)]
        vp = v & 32767
        vr_s = jnp.where(valid, v >> 15, 0)
        t = plsc.load_gather(table, [vr_s])
        w = valid & (t == vp)
        wi = w.astype(jnp.int32)
        slots = woff + plsc.cumsum(wi) - 1
        slots = jnp.where(w, slots, 0)
        plsc.store_scatter(wl, [slots], v, mask=w)
        return woff + jnp.sum(wi)

    m = lax.fori_loop(0, nv, wk, jnp.int32(0))

    mpad = lax.div(m + (SCCH - 1), jnp.int32(SCCH)) * SCCH

    @pl.when(m > 0)
    def _pad():
        def padk(k, carry):
            idxs = m + k * L + iota
            mk = idxs < mpad
            src = idxs % m
            vals = plsc.load_gather(wl, [src])
            idxs_s = jnp.where(mk, idxs, 0)
            plsc.store_scatter(wl, [idxs_s], vals, mask=mk)
            return carry

        lax.fori_loop(0, SCCH // L, padk, 0)

    def wout(i, carry):
        pltpu.sync_copy(wl.at[pl.ds(i * IDC, IDC)],
                        wl_hbm.at[wid].at[pl.ds(i * IDC, IDC)])
        return carry

    lax.fori_loop(0, (mpad + (IDC - 1)) // IDC, wout, 0)
    for j in range(MD // L):
        idc[pl.ds(j * L, L)] = jnp.broadcast_to(mpad, (L,))
    pltpu.sync_copy(idc.at[pl.ds(0, MD)], cnt_hbm.at[wid])


def _sc_winners(ids, memory, base_ref):
    mesh = plsc.VectorSubcoreMesh(core_axis_name="c", subcore_axis_name="s")
    fn = pl.kernel(
        _winners_body,
        out_type=(
            jax.ShapeDtypeStruct((NW, TWO_B), jnp.int32),
            jax.ShapeDtypeStruct((NW, MD), jnp.int32),
        ),
        mesh=mesh,
        compiler_params=pltpu.CompilerParams(needs_layout_passes=False),
        scratch_types=[
            pltpu.VMEM((IDC,), jnp.int32),
            pltpu.VMEM((TWO_B + L,), jnp.int32),
            pltpu.VMEM((TBL,), jnp.int32),
            pltpu.VMEM((CCH, MD), jnp.float32),
            pltpu.VMEM((CCH, MD), jnp.float32),
            pltpu.SemaphoreType.DMA,
            pltpu.SemaphoreType.DMA,
            pltpu.SemaphoreType.DMA,
            pltpu.SemaphoreType.DMA,
        ],
    )
    return fn(ids, memory, base_ref)


def _apply_body(base_hbm, upd_hbm, wl_hbm, cnt_hbm,
                cbuf, wbuf, sg0, sp0, r0, sg1, sp1, r1, sem0, sem1):
    wid = lax.axis_index("s") * NC + lax.axis_index("c")
    pltpu.sync_copy(cnt_hbm.at[wid], cbuf)
    cv = cbuf[pl.ds(0, L)]
    mpad = cv[0]
    nch = mpad // SCCH

    def stage_and_fire(c, sg, sp, rows, sem):
        pltpu.sync_copy(wl_hbm.at[wid].at[pl.ds(c * SCCH, SCCH)], wbuf)

        def upk(k, carry):
            v = wbuf[pl.ds(k * L, L)]
            sp[pl.ds(k * L, L)] = v & 32767
            sg[pl.ds(k * L, L)] = ((v >> 15) << 5) | wid
            return carry

        lax.fori_loop(0, SCCH // L, upk, 0)
        pltpu.async_copy(upd_hbm.at[sp], rows, sem)

    def drain_and_scatter(sg, sp, rows, sem):
        pltpu.make_async_copy(upd_hbm.at[sp], rows, sem).wait()
        pltpu.async_copy(rows, base_hbm.at[sg], sem).wait()

    @pl.when(nch > 0)
    def _go():
        stage_and_fire(0, sg0, sp0, r0, sem0)

        def sck(c, carry):
            @pl.when(c % 2 == 0)
            def _even():
                @pl.when(c + 1 < nch)
                def _pf():
                    stage_and_fire(c + 1, sg1, sp1, r1, sem1)
                drain_and_scatter(sg0, sp0, r0, sem0)

            @pl.when(c % 2 == 1)
            def _odd():
                @pl.when(c + 1 < nch)
                def _pf():
                    stage_and_fire(c + 1, sg0, sp0, r0, sem0)
                drain_and_scatter(sg1, sp1, r1, sem1)

            return carry

        lax.fori_loop(0, nch, sck, 0)


def _sc_apply(base_ref, upds, wl, cnt):
    mesh = plsc.VectorSubcoreMesh(core_axis_name="c", subcore_axis_name="s")
    fn = pl.kernel(
        _apply_body,
        out_type=(),
        mesh=mesh,
        compiler_params=pltpu.CompilerParams(needs_layout_passes=False),
        scratch_types=[
            pltpu.VMEM((MD,), jnp.int32),
            pltpu.VMEM((SCCH,), jnp.int32),
            pltpu.VMEM((SCCH,), jnp.int32),
            pltpu.VMEM((SCCH,), jnp.int32),
            pltpu.VMEM((SCCH, MD), jnp.float32),
            pltpu.VMEM((SCCH,), jnp.int32),
            pltpu.VMEM((SCCH,), jnp.int32),
            pltpu.VMEM((SCCH, MD), jnp.float32),
            pltpu.SemaphoreType.DMA,
            pltpu.SemaphoreType.DMA,
        ],
    )
    fn(base_ref, upds, wl, cnt)


def _dense_body(g_ref, se_ref, de_ref, ef_ref, ts_ref, tw_ref, tb_ref,
                evm_ref, evt_ref, as_ref, ao_ref, ae_ref, at_ref,
                ca_ref, ce_ref, ct_ref, cw2_ref, ou_ref, on_ref,
                eb_ref, ab_ref, c1b_ref, c2b_ref, ob_ref,
                out_ref, upd_ref):
    f32 = jnp.float32
    bf16 = jnp.bfloat16

    def dot(a, b):
        return lax.dot_general(a.astype(bf16), b, (((1,), (0,)), ((), ())),
                               preferred_element_type=f32)

    te = jnp.cos(ts_ref[...] * tw_ref[...] + tb_ref[...])
    ef = ef_ref[...]
    sm = g_ref[0]
    dm = g_ref[1]

    sh_e = dot(te, evt_ref[...]) + eb_ref[...]
    s_ev = jnp.tanh(dot(sm, evm_ref[...]) + sh_e)
    d_ev = jnp.tanh(dot(dm, evm_ref[...]) + sh_e)

    sh_a = dot(ef, ae_ref[...]) + dot(te, at_ref[...]) + ab_ref[...]
    s_as = jnp.tanh(dot(s_ev, as_ref[...]) + dot(d_ev, ao_ref[...]) + sh_a)
    d_as = jnp.tanh(dot(d_ev, as_ref[...]) + dot(s_ev, ao_ref[...]) + sh_a)

    sh_c = dot(ef, ce_ref[...]) + dot(te, ct_ref[...]) + c1b_ref[...]
    s_c1 = jnp.maximum(dot(s_as, ca_ref[...]) + sh_c, 0.0)
    d_c1 = jnp.maximum(dot(d_as, ca_ref[...]) + sh_c, 0.0)
    s_cm = jnp.tanh(dot(s_c1, cw2_ref[...]) + c2b_ref[...])
    d_cm = jnp.tanh(dot(d_c1, cw2_ref[...]) + c2b_ref[...])

    u_s = s_ev + s_cm
    u_d = d_ev + d_cm
    upd_ref[0] = u_s
    upd_ref[1] = u_d
    out_ref[0] = dot(u_s, ou_ref[...]) + dot(se_ref[...], on_ref[...]) + ob_ref[...]
    out_ref[1] = dot(u_d, ou_ref[...]) + dot(de_ref[...], on_ref[...]) + ob_ref[...]


BLK = 2048


def _tc_dense(g3, semb, demb, ef, ts2, tw, tb, weights):
    f32 = jnp.float32
    grid = (B // BLK,)

    def full(shape):
        return pl.BlockSpec(shape, lambda g: tuple(0 for _ in shape))

    in_specs = [
        pl.BlockSpec((2, BLK, MD), lambda g: (0, g, 0)),
        pl.BlockSpec((BLK, MD), lambda g: (g, 0)),
        pl.BlockSpec((BLK, MD), lambda g: (g, 0)),
        pl.BlockSpec((BLK, ED), lambda g: (g, 0)),
        pl.BlockSpec((BLK, 1), lambda g: (g, 0)),
        full((1, MD)), full((1, MD)),
        full((MD, MD)), full((MD, MD)),
        full((MD, MD)), full((MD, MD)), full((ED, MD)), full((MD, MD)),
        full((MD, MD)), full((ED, MD)), full((MD, MD)), full((MD, MD)),
        full((MD, MD)), full((MD, MD)),
        full((1, MD)), full((1, MD)), full((1, MD)), full((1, MD)), full((1, MD)),
    ]
    out_specs = [
        pl.BlockSpec((2, BLK, MD), lambda g: (0, g, 0)),
        pl.BlockSpec((2, BLK, MD), lambda g: (0, g, 0)),
    ]
    out_shape = [
        jax.ShapeDtypeStruct((2, B, MD), f32),
        jax.ShapeDtypeStruct((2, B, MD), f32),
    ]
    return pl.pallas_call(
        _dense_body,
        grid=grid,
        in_specs=in_specs,
        out_specs=out_specs,
        out_shape=out_shape,
    )(g3, semb, demb, ef, ts2, tw, tb, *weights)


CPB = 4520


def _copy_body(src_ref, dst_ref):
    dst_ref[...] = src_ref[...]


def _tc_copy(memory):
    return pl.pallas_call(
        _copy_body,
        grid=(CT // CPB,),
        in_specs=[pl.BlockSpec((CPB, MD), lambda g: (g, 0))],
        out_specs=pl.BlockSpec((CPB, MD), lambda g: (g, 0)),
        out_shape=jax.ShapeDtypeStruct((N_NODES, MD), jnp.float32),
    )(memory)


def kernel(src_node_embeddings, dst_node_embeddings, src_node_ids,
           dst_node_ids, edge_features, timestamps, memory,
           time_w, time_b, evo_w, evo_b, assoc_w, assoc_b,
           comm_w1, comm_b1, comm_w2, comm_b2, out_w, out_b):
    f32 = jnp.float32
    ids = jnp.concatenate([src_node_ids.astype(jnp.int32),
                           dst_node_ids.astype(jnp.int32)])

    g3 = _sc_gather(memory, ids)

    ts2 = timestamps.reshape(B, 1)
    pad_t = MD - TD
    tw = jnp.pad(time_w, (0, pad_t)).reshape(1, MD)
    tb = jnp.pad(time_b, (0, pad_t)).reshape(1, MD)
    evm = evo_w[:MD]
    evt = jnp.pad(evo_w[MD:], ((0, pad_t), (0, 0)))
    a_s = assoc_w[:MD]
    a_o = assoc_w[MD:2 * MD]
    a_e = assoc_w[2 * MD:2 * MD + ED]
    a_t = jnp.pad(assoc_w[2 * MD + ED:], ((0, pad_t), (0, 0)))
    c_a = comm_w1[:MD]
    c_e = comm_w1[MD:MD + ED]
    c_t = jnp.pad(comm_w1[MD + ED:], ((0, pad_t), (0, 0)))
    o_u = out_w[:MD]
    o_n = out_w[MD:]
    bf16 = jnp.bfloat16
    weights = tuple(w.astype(bf16) for w in
                    (evm, evt, a_s, a_o, a_e, a_t, c_a, c_e, c_t, comm_w2,
                     o_u, o_n)) + (
               evo_b.reshape(1, MD), assoc_b.reshape(1, MD),
               comm_b1.reshape(1, MD), comm_b2.reshape(1, MD),
               out_b.reshape(1, MD))

    base = _tc_copy(memory)
    base_ref = jax.new_ref(base)
    wl, cnt = _sc_winners(ids, memory, base_ref)
    outp3, upd3 = _tc_dense(g3, src_node_embeddings, dst_node_embeddings,
                            edge_features, ts2, tw, tb, weights)

    output = outp3.reshape(TWO_B, MD)
    upds = upd3.reshape(TWO_B, MD)
    _sc_apply(base_ref, upds, wl, cnt)
    new_memory = base_ref[...]
    return output, new_memory
